# Initial kernel scaffold; baseline (speedup 1.0000x reference)
import jax
import jax.numpy as jnp
from jax.experimental import pallas as pl


def kernel(idx, table):
    raise NotImplementedError("write your pallas kernel here")


# speedup vs baseline: 1.0195x; 1.0195x over previous
import functools

import jax
import jax.numpy as jnp
from jax import lax
from jax.experimental import pallas as pl
from jax.experimental.pallas import tpu as pltpu
from jax.experimental.pallas import tpu_sc as plsc

VOCAB = 1000
NC, NS = 2, 16
NW = NC * NS
N_TOKENS = 1024 * 50
PER_W = N_TOKENS // NW
CHUNK = 80
N_CHUNKS = PER_W // CHUNK

_mesh = plsc.VectorSubcoreMesh(core_axis_name="c", subcore_axis_name="s")


@functools.partial(
    pl.kernel,
    out_type=jax.ShapeDtypeStruct((N_TOKENS, VOCAB), jnp.float32),
    mesh=_mesh,
    scratch_types=[
        pltpu.VMEM((N_CHUNKS, CHUNK), jnp.int32),
        pltpu.VMEM((CHUNK, VOCAB), jnp.float32),
        pltpu.SemaphoreType.DMA,
    ],
    compiler_params=pltpu.CompilerParams(use_tc_tiling_on_sc=False),
)
def _sc_gather(table_hbm, idx_hbm, out_hbm, idx_v, rows_v, sem):
    wid = lax.axis_index("s") * NC + lax.axis_index("c")
    base = wid * PER_W
    pltpu.sync_copy(idx_hbm.at[wid], idx_v)

    def body(j, carry):
        pltpu.async_copy(table_hbm.at[idx_v.at[j]], rows_v, sem).wait()
        pltpu.sync_copy(rows_v, out_hbm.at[pl.ds(base + j * CHUNK, CHUNK)])
        return carry

    lax.fori_loop(0, N_CHUNKS, body, 0)


def kernel(idx, table):
    B, T = idx.shape
    idx_grouped = idx.reshape(NW, N_CHUNKS, CHUNK).astype(jnp.int32)
    out = _sc_gather(table, idx_grouped)
    return out.reshape(B, T, VOCAB)


# speedup vs baseline: 1.0303x; 1.0106x over previous
import functools

import jax
import jax.numpy as jnp
from jax import lax
from jax.experimental import pallas as pl
from jax.experimental.pallas import tpu as pltpu
from jax.experimental.pallas import tpu_sc as plsc

VOCAB = 1000
NC, NS = 2, 16
NW = NC * NS
N_TOKENS = 1024 * 50
PER_W = N_TOKENS // NW
CHUNK = 64
N_CHUNKS = PER_W // CHUNK

_mesh = plsc.VectorSubcoreMesh(core_axis_name="c", subcore_axis_name="s")


@functools.partial(
    pl.kernel,
    out_type=jax.ShapeDtypeStruct((N_TOKENS, VOCAB), jnp.float32),
    mesh=_mesh,
    scratch_types=[
        pltpu.VMEM((N_CHUNKS, CHUNK), jnp.int32),
        pltpu.VMEM((2, CHUNK, VOCAB), jnp.float32),
        pltpu.SemaphoreType.DMA((2,)),
        pltpu.SemaphoreType.DMA((2,)),
    ],
    compiler_params=pltpu.CompilerParams(use_tc_tiling_on_sc=False),
)
def _sc_gather(table_hbm, idx_hbm, out_hbm, idx_v, rows_v, gsem, wsem):
    wid = lax.axis_index("s") * NC + lax.axis_index("c")
    base = wid * PER_W
    pltpu.sync_copy(idx_hbm.at[wid], idx_v)

    def gather(j, b):
        return pltpu.make_async_copy(
            table_hbm.at[idx_v.at[j]], rows_v.at[b], gsem.at[b])

    def write(j, b):
        return pltpu.make_async_copy(
            rows_v.at[b], out_hbm.at[pl.ds(base + j * CHUNK, CHUNK)],
            wsem.at[b])

    gather(0, 0).start()
    gather(1, 1).start()

    def body(j, carry):
        b = lax.rem(j, 2)
        gather(j, b).wait()
        write(j, b).start()

        @pl.when(j + 2 < N_CHUNKS)
        def _():
            write(j, b).wait()
            gather(j + 2, b).start()

        return carry

    lax.fori_loop(0, N_CHUNKS, body, 0)
    write(N_CHUNKS - 2, (N_CHUNKS - 2) % 2).wait()
    write(N_CHUNKS - 1, (N_CHUNKS - 1) % 2).wait()


def kernel(idx, table):
    B, T = idx.shape
    idx_grouped = idx.reshape(NW, N_CHUNKS, CHUNK).astype(jnp.int32)
    out = _sc_gather(table, idx_grouped)
    return out.reshape(B, T, VOCAB)


# speedup vs baseline: 1.0307x; 1.0004x over previous
import functools

import jax
import jax.numpy as jnp
from jax import lax
from jax.experimental import pallas as pl
from jax.experimental.pallas import tpu as pltpu
from jax.experimental.pallas import tpu_sc as plsc

VOCAB = 1000
B, T = 1024, 50
NC, NS = 2, 16
NW = NC * NS
NVC, NBC = 8, 8
V0S = (0, 128, 256, 384, 512, 640, 768, 872)
UNITS = T * NVC * NBC
PER_W = UNITS // NW

_mesh = plsc.VectorSubcoreMesh(core_axis_name="c", subcore_axis_name="s")


@functools.partial(
    pl.kernel,
    out_type=jax.ShapeDtypeStruct((T, VOCAB, B), jnp.float32),
    mesh=_mesh,
    scratch_types=[
        pltpu.VMEM((PER_W, 128), jnp.int32),
        pltpu.VMEM((2, 128, 128), jnp.float32),
        pltpu.VMEM((2, 128, 128), jnp.float32),
        pltpu.SemaphoreType.DMA((2,)),
        pltpu.SemaphoreType.DMA((2,)),
    ],
    compiler_params=pltpu.CompilerParams(
        use_tc_tiling_on_sc=True, needs_layout_passes=False),
)
def _sc_gather_t(table_r, idx_u, out_t, idx_v, rows_v, outf_v, gsem, wsem):
    wid = lax.axis_index("s") * NC + lax.axis_index("c")
    pltpu.sync_copy(idx_u.at[wid], idx_v)

    iota = lax.iota(jnp.int32, 16)

    def decode(u):
        unit = wid * PER_W + u
        t = unit // (NVC * NBC)
        r = lax.rem(unit, NVC * NBC)
        vc = r // NBC
        bc = lax.rem(r, NBC)
        return t, vc, bc

    def gather(u, s):
        return pltpu.make_async_copy(
            table_r.at[idx_v.at[u]], rows_v.at[s], gsem.at[s])

    def write(u, s):
        t, vc, bc = decode(u)
        v0 = pl.multiple_of(jnp.where(vc == NVC - 1, 872, vc * 128), 8)
        b0 = pl.multiple_of(bc * 128, 128)
        return pltpu.make_async_copy(
            outf_v.at[s],
            out_t.at[t, pl.ds(v0, 128), pl.ds(b0, 128)],
            wsem.at[s])

    def transpose(s):
        sidx = jnp.full((16,), s, jnp.int32)

        @plsc.parallel_loop(0, 128, 1, unroll=8)
        def _(v):
            vfull = jnp.full((16,), v, jnp.int32)
            for bb in range(8):
                vals = plsc.load_gather(
                    rows_v, [sidx, iota + (16 * bb), vfull])
                outf_v[s, v, pl.ds(16 * bb, 16)] = vals

    gather(0, 0).start()
    gather(1, 1).start()

    def body(u, carry):
        s = lax.rem(u, 2)

        @pl.when(u >= 2)
        def _():
            write(u - 2, s).wait()

        gather(u, s).wait()
        transpose(s)

        @pl.when(u + 2 < PER_W)
        def _():
            gather(u + 2, s).start()

        write(u, s).start()
        return carry

    lax.fori_loop(0, PER_W, body, 0)
    write(PER_W - 2, (PER_W - 2) % 2).wait()
    write(PER_W - 1, (PER_W - 1) % 2).wait()


def kernel(idx, table):
    table_r = jnp.stack([lax.slice(table, (0, v0), (VOCAB, v0 + 128))
                         for v0 in V0S])
    table_r = table_r.reshape(NVC * VOCAB, 128)

    idx_t = idx.astype(jnp.int32).T.reshape(T, 1, NBC, 128)
    idx_u = idx_t + (VOCAB * jnp.arange(NVC, dtype=jnp.int32))[None, :, None, None]
    idx_u = idx_u.reshape(NW, PER_W, 128)

    out_t = _sc_gather_t(table_r, idx_u)
    return jnp.transpose(out_t, (2, 0, 1))
